# baseline (speedup 1.0000x reference)
import functools

import jax
import jax.numpy as jnp
from jax import lax
from jax.experimental import pallas as pl
from jax.experimental.pallas import tpu as pltpu
from jax.experimental.pallas import tpu_sc as plsc

BATCH = 16384
F = 100
D = 16
NCLS = 2
V = 100000
NC, NS = 2, 16
NW = NC * NS
PER_W = BATCH // NW
CB = 32
NCH = PER_W // CB
BV = 4096


def _tbl_body(w1_ref, b1_ref, w2_ref, b2_ref, wo_ref, bo_ref, emb_ref,
              packed_ref, beff_ref):
    f32 = jnp.float32
    hi = lax.Precision.HIGHEST
    mT = lax.dot_general(wo_ref[...], w2_ref[...], (((0,), (1,)), ((), ())),
                         preferred_element_type=f32, precision=hi)
    weffT = lax.dot_general(mT, w1_ref[...], (((1,), (1,)), ((), ())),
                            preferred_element_type=f32, precision=hi)
    beff = (lax.dot_general(b1_ref[...], mT, (((1,), (1,)), ((), ())),
                            preferred_element_type=f32, precision=hi)
            + jnp.dot(b2_ref[...], wo_ref[...], preferred_element_type=f32,
                      precision=hi)
            + bo_ref[...])
    z = lax.dot_general(emb_ref[...], weffT, (((1,), (1,)), ((), ())),
                        preferred_element_type=f32, precision=hi)
    lo = lax.bitcast_convert_type(
        z[:, 0:1].astype(jnp.bfloat16), jnp.uint16).astype(jnp.uint32)
    hi16 = lax.bitcast_convert_type(
        z[:, 1:2].astype(jnp.bfloat16), jnp.uint16).astype(jnp.uint32)
    packed = lax.bitcast_convert_type(lo | (hi16 << 16), jnp.int32)
    packed_ref[...] = jnp.squeeze(packed, -1)
    beff_ref[...] = jnp.concatenate(
        [beff, jnp.zeros((1, D - NCLS), f32)], axis=1) + jnp.zeros((8, D), f32)


def _packed_table(W1, b1, W2, b2, Wout, bout, emb):
    full = lambda shape: pl.BlockSpec(shape, lambda i: (0,) * len(shape))
    return pl.pallas_call(
        _tbl_body,
        grid=(pl.cdiv(V, BV),),
        in_specs=[
            full((D, 64)), full((1, 64)), full((64, D)), full((1, D)),
            full((D, NCLS)), full((1, NCLS)),
            pl.BlockSpec((BV, D), lambda i: (i, 0)),
        ],
        out_specs=[
            pl.BlockSpec((BV,), lambda i: (i,)),
            pl.BlockSpec((8, D), lambda i: (0, 0)),
        ],
        out_shape=[
            jax.ShapeDtypeStruct((V,), jnp.int32),
            jax.ShapeDtypeStruct((8, D), jnp.float32),
        ],
    )(W1, b1.reshape(1, -1), W2, b2.reshape(1, -1), Wout, bout.reshape(1, -1),
      emb)


def _sc_body(x_hbm, tbl_hbm, hw_hbm, out_hbm,
             idx_v, ew_v, out_v, hw_v, isem, osem):
    c = lax.axis_index("c")
    s = lax.axis_index("s")
    wid = s * NC + c
    row0 = wid * PER_W

    pltpu.sync_copy(hw_hbm, hw_v)
    pltpu.sync_copy(tbl_hbm, ew_v)

    def idx_copy(i, buf):
        return pltpu.make_async_copy(
            x_hbm.at[pl.ds(row0 + i * CB, CB), :], idx_v.at[buf],
            isem.at[buf])

    def out_copy(i, buf):
        return pltpu.make_async_copy(
            out_v.at[buf], out_hbm.at[pl.ds(row0 + i * CB, CB)], osem.at[buf])

    idx_copy(0, 0).start()
    idx_copy(1, 1).start()

    lanes = jnp.arange(16, dtype=jnp.int32)
    beff = hw_v[0, :]
    himask = jnp.int32(-65536)

    def outer(i0, carry):
        for b in range(2):
            i = i0 * 2 + b
            idx_copy(i, b).wait()

            @pl.when(i >= 2)
            def _():
                out_copy(i - 2, b).wait()

            for g in range(CB // 16):
                rows = lanes + g * 16
                z0 = jnp.full((16,), beff[0], jnp.float32)
                z1 = jnp.full((16,), beff[1], jnp.float32)
                for f in range(F):
                    xi = plsc.load_gather(
                        idx_v.at[b], [rows, jnp.full((16,), f, jnp.int32)])
                    w = plsc.load_gather(ew_v, [xi])
                    z0 = z0 + plsc.bitcast(w << 16, jnp.float32)
                    z1 = z1 + plsc.bitcast(w & himask, jnp.float32)
                s0 = 1.0 / (1.0 + jnp.exp(-z0))
                s1 = 1.0 / (1.0 + jnp.exp(-z1))
                plsc.store_scatter(
                    out_v.at[b], [rows, jnp.full((16,), 0, jnp.int32)], s0)
                plsc.store_scatter(
                    out_v.at[b], [rows, jnp.full((16,), 1, jnp.int32)], s1)

            @pl.when(i + 2 < NCH)
            def _():
                idx_copy(i + 2, b).start()

            out_copy(i, b).start()
        return carry

    lax.fori_loop(0, NCH // 2, outer, 0)
    out_copy(NCH - 2, 0).wait()
    out_copy(NCH - 1, 1).wait()


_sc_main = functools.partial(
    pl.kernel,
    out_type=jax.ShapeDtypeStruct((BATCH, NCLS), jnp.float32),
    mesh=plsc.VectorSubcoreMesh(core_axis_name="c", subcore_axis_name="s"),
    scratch_types=[
        pltpu.VMEM((2, CB, F), jnp.int32),
        pltpu.VMEM((V,), jnp.int32),
        pltpu.VMEM((2, CB, NCLS), jnp.float32),
        pltpu.VMEM((8, D), jnp.float32),
        pltpu.SemaphoreType.DMA((2,)),
        pltpu.SemaphoreType.DMA((2,)),
    ],
    compiler_params=pltpu.CompilerParams(use_tc_tiling_on_sc=False,
                                         needs_layout_passes=False),
)(_sc_body)


def kernel(x, emb, W1, b1, W2, b2, Wout, bout):
    tbl, hw = _packed_table(W1, b1, W2, b2, Wout, bout, emb)
    return _sc_main(x, tbl, hw)


# speedup vs baseline: 1.3299x; 1.3299x over previous
import functools

import jax
import jax.numpy as jnp
from jax import lax
from jax.experimental import pallas as pl
from jax.experimental.pallas import tpu as pltpu
from jax.experimental.pallas import tpu_sc as plsc

BATCH = 16384
F = 100
D = 16
NCLS = 2
V = 100000
NC, NS = 2, 16
NW = NC * NS
PER_W = BATCH // NW
CB = 32
NCH = PER_W // CB
BV = 12800


def _tbl_body(w1_ref, b1_ref, w2_ref, b2_ref, wo_ref, bo_ref, emb_ref,
              packed_ref, beff_ref):
    f32 = jnp.float32
    hi = lax.Precision.HIGHEST
    mT = lax.dot_general(wo_ref[...], w2_ref[...], (((0,), (1,)), ((), ())),
                         preferred_element_type=f32, precision=hi)
    weffT = lax.dot_general(mT, w1_ref[...], (((1,), (1,)), ((), ())),
                            preferred_element_type=f32, precision=hi)
    beff = (lax.dot_general(b1_ref[...], mT, (((1,), (1,)), ((), ())),
                            preferred_element_type=f32, precision=hi)
            + jnp.dot(b2_ref[...], wo_ref[...], preferred_element_type=f32,
                      precision=hi)
            + bo_ref[...])
    z = lax.dot_general(weffT, emb_ref[...], (((1,), (1,)), ((), ())),
                        preferred_element_type=f32, precision=hi)
    lo = lax.bitcast_convert_type(
        z[0:1, :].astype(jnp.bfloat16), jnp.uint16).astype(jnp.uint32)
    hi16 = lax.bitcast_convert_type(
        z[1:2, :].astype(jnp.bfloat16), jnp.uint16).astype(jnp.uint32)
    packed_ref[...] = lax.bitcast_convert_type(lo | (hi16 << 16), jnp.int32)
    beff_ref[...] = jnp.concatenate(
        [beff, jnp.zeros((1, D - NCLS), f32)], axis=1) + jnp.zeros((8, D), f32)


def _packed_table(W1, b1, W2, b2, Wout, bout, emb):
    full = lambda shape: pl.BlockSpec(shape, lambda i: (0,) * len(shape))
    return pl.pallas_call(
        _tbl_body,
        grid=(pl.cdiv(V, BV),),
        in_specs=[
            full((D, 64)), full((1, 64)), full((64, D)), full((1, D)),
            full((D, NCLS)), full((1, NCLS)),
            pl.BlockSpec((BV, D), lambda i: (i, 0)),
        ],
        out_specs=[
            pl.BlockSpec((1, BV), lambda i: (0, i)),
            pl.BlockSpec((8, D), lambda i: (0, 0)),
        ],
        out_shape=[
            jax.ShapeDtypeStruct((1, V), jnp.int32),
            jax.ShapeDtypeStruct((8, D), jnp.float32),
        ],
    )(W1, b1.reshape(1, -1), W2, b2.reshape(1, -1), Wout, bout.reshape(1, -1),
      emb)


def _sc_body(x_hbm, tbl_hbm, hw_hbm, out_hbm,
             idx_v, ew_v, out_v, hw_v, isem, osem):
    c = lax.axis_index("c")
    s = lax.axis_index("s")
    wid = s * NC + c
    row0 = wid * PER_W

    pltpu.sync_copy(hw_hbm, hw_v)
    pltpu.sync_copy(tbl_hbm.at[0], ew_v)

    def idx_copy(i, buf):
        return pltpu.make_async_copy(
            x_hbm.at[pl.ds(row0 + i * CB, CB), :], idx_v.at[buf],
            isem.at[buf])

    def out_copy(i, buf):
        return pltpu.make_async_copy(
            out_v.at[buf], out_hbm.at[pl.ds(row0 + i * CB, CB)], osem.at[buf])

    idx_copy(0, 0).start()
    idx_copy(1, 1).start()

    lanes = jnp.arange(16, dtype=jnp.int32)
    beff = hw_v[0, :]
    himask = jnp.int32(-65536)

    def outer(i0, carry):
        for b in range(2):
            i = i0 * 2 + b
            idx_copy(i, b).wait()

            @pl.when(i >= 2)
            def _():
                out_copy(i - 2, b).wait()

            for g in range(CB // 16):
                rows = lanes + g * 16
                z0 = jnp.full((16,), beff[0], jnp.float32)
                z1 = jnp.full((16,), beff[1], jnp.float32)
                for f in range(F):
                    xi = plsc.load_gather(
                        idx_v.at[b], [rows, jnp.full((16,), f, jnp.int32)])
                    w = plsc.load_gather(ew_v, [xi])
                    z0 = z0 + plsc.bitcast(w << 16, jnp.float32)
                    z1 = z1 + plsc.bitcast(w & himask, jnp.float32)
                s0 = 1.0 / (1.0 + jnp.exp(-z0))
                s1 = 1.0 / (1.0 + jnp.exp(-z1))
                plsc.store_scatter(
                    out_v.at[b], [rows, jnp.full((16,), 0, jnp.int32)], s0)
                plsc.store_scatter(
                    out_v.at[b], [rows, jnp.full((16,), 1, jnp.int32)], s1)

            @pl.when(i + 2 < NCH)
            def _():
                idx_copy(i + 2, b).start()

            out_copy(i, b).start()
        return carry

    lax.fori_loop(0, NCH // 2, outer, 0)
    out_copy(NCH - 2, 0).wait()
    out_copy(NCH - 1, 1).wait()


_sc_main = functools.partial(
    pl.kernel,
    out_type=jax.ShapeDtypeStruct((BATCH, NCLS), jnp.float32),
    mesh=plsc.VectorSubcoreMesh(core_axis_name="c", subcore_axis_name="s"),
    scratch_types=[
        pltpu.VMEM((2, CB, F), jnp.int32),
        pltpu.VMEM((V,), jnp.int32),
        pltpu.VMEM((2, CB, NCLS), jnp.float32),
        pltpu.VMEM((8, D), jnp.float32),
        pltpu.SemaphoreType.DMA((2,)),
        pltpu.SemaphoreType.DMA((2,)),
    ],
    compiler_params=pltpu.CompilerParams(use_tc_tiling_on_sc=False,
                                         needs_layout_passes=False),
)(_sc_body)


def kernel(x, emb, W1, b1, W2, b2, Wout, bout):
    tbl, hw = _packed_table(W1, b1, W2, b2, Wout, bout, emb)
    return _sc_main(x, tbl, hw)


# speedup vs baseline: 1.5469x; 1.1631x over previous
import functools

import jax
import jax.numpy as jnp
from jax import lax
from jax.experimental import pallas as pl
from jax.experimental.pallas import tpu as pltpu
from jax.experimental.pallas import tpu_sc as plsc

BATCH = 16384
F = 100
D = 16
NCLS = 2
V = 100000
NC, NS = 2, 16
NW = NC * NS
PER_W = BATCH // NW
CB = 32
NCH = PER_W // CB
BV = 12288


def _tbl_body(w1_ref, b1_ref, w2_ref, b2_ref, wo_ref, bo_ref, emb_ref,
              packed_ref, beff_ref):
    f32 = jnp.float32
    hi = lax.Precision.HIGHEST
    mT = lax.dot_general(wo_ref[...], w2_ref[...], (((0,), (1,)), ((), ())),
                         preferred_element_type=f32, precision=hi)
    weffT = lax.dot_general(mT, w1_ref[...], (((1,), (1,)), ((), ())),
                            preferred_element_type=f32, precision=hi)
    beff = (lax.dot_general(b1_ref[...], mT, (((1,), (1,)), ((), ())),
                            preferred_element_type=f32, precision=hi)
            + jnp.dot(b2_ref[...], wo_ref[...], preferred_element_type=f32,
                      precision=hi)
            + bo_ref[...])
    bf16 = jnp.bfloat16
    e = emb_ref[...]
    eh = e.astype(bf16)
    el = (e - eh.astype(f32)).astype(bf16)
    wh = weffT.astype(bf16)
    wl = (weffT - wh.astype(f32)).astype(bf16)
    dd = lambda a, b: lax.dot_general(
        a, b, (((1,), (1,)), ((), ())), preferred_element_type=f32)
    z = dd(wh, eh) + (dd(wh, el) + dd(wl, eh))
    lo = lax.bitcast_convert_type(
        z[0:1, :].astype(jnp.bfloat16), jnp.uint16).astype(jnp.uint32)
    hi16 = lax.bitcast_convert_type(
        z[1:2, :].astype(jnp.bfloat16), jnp.uint16).astype(jnp.uint32)
    packed_ref[...] = jnp.squeeze(
        lax.bitcast_convert_type(lo | (hi16 << 16), jnp.int32), 0)
    beff_ref[...] = jnp.concatenate(
        [jnp.squeeze(beff, 0), jnp.zeros((126,), f32)], axis=0)


def _packed_table(W1, b1, W2, b2, Wout, bout, emb):
    full = lambda shape: pl.BlockSpec(shape, lambda i: (0,) * len(shape))
    return pl.pallas_call(
        _tbl_body,
        grid=(pl.cdiv(V, BV),),
        in_specs=[
            full((D, 64)), full((1, 64)), full((64, D)), full((1, D)),
            full((D, NCLS)), full((1, NCLS)),
            pl.BlockSpec((BV, D), lambda i: (i, 0)),
        ],
        out_specs=[
            pl.BlockSpec((BV,), lambda i: (i,)),
            pl.BlockSpec((128,), lambda i: (0,)),
        ],
        out_shape=[
            jax.ShapeDtypeStruct((V,), jnp.int32),
            jax.ShapeDtypeStruct((128,), jnp.float32),
        ],
    )(W1, b1.reshape(1, -1), W2, b2.reshape(1, -1), Wout, bout.reshape(1, -1),
      emb)


def _sc_body(x_hbm, tbl_hbm, hw_hbm, out_hbm,
             idx_v, ew_v, out_v, hw_v, isem, osem):
    c = lax.axis_index("c")
    s = lax.axis_index("s")
    wid = s * NC + c
    row0 = wid * PER_W

    pltpu.sync_copy(hw_hbm, hw_v)
    pltpu.sync_copy(tbl_hbm, ew_v)

    def idx_copy(i, buf):
        return pltpu.make_async_copy(
            x_hbm.at[pl.ds(row0 + i * CB, CB), :], idx_v.at[buf],
            isem.at[buf])

    def out_copy(i, buf):
        return pltpu.make_async_copy(
            out_v.at[buf], out_hbm.at[pl.ds(row0 + i * CB, CB)], osem.at[buf])

    idx_copy(0, 0).start()
    idx_copy(1, 1).start()

    lanes = jnp.arange(16, dtype=jnp.int32)
    beff = hw_v[pl.ds(0, 16)]
    himask = jnp.int32(-65536)

    def outer(i0, carry):
        for b in range(2):
            i = i0 * 2 + b
            idx_copy(i, b).wait()

            @pl.when(i >= 2)
            def _():
                out_copy(i - 2, b).wait()

            for g in range(CB // 16):
                rows = lanes + g * 16
                z0 = jnp.full((16,), beff[0], jnp.float32)
                z1 = jnp.full((16,), beff[1], jnp.float32)
                for f in range(F):
                    xi = plsc.load_gather(
                        idx_v.at[b], [rows, jnp.full((16,), f, jnp.int32)])
                    w = plsc.load_gather(ew_v, [xi])
                    z0 = z0 + plsc.bitcast(w << 16, jnp.float32)
                    z1 = z1 + plsc.bitcast(w & himask, jnp.float32)
                s0 = 1.0 / (1.0 + jnp.exp(-z0))
                s1 = 1.0 / (1.0 + jnp.exp(-z1))
                plsc.store_scatter(
                    out_v.at[b], [rows, jnp.full((16,), 0, jnp.int32)], s0)
                plsc.store_scatter(
                    out_v.at[b], [rows, jnp.full((16,), 1, jnp.int32)], s1)

            @pl.when(i + 2 < NCH)
            def _():
                idx_copy(i + 2, b).start()

            out_copy(i, b).start()
        return carry

    lax.fori_loop(0, NCH // 2, outer, 0)
    out_copy(NCH - 2, 0).wait()
    out_copy(NCH - 1, 1).wait()


_sc_main = functools.partial(
    pl.kernel,
    out_type=jax.ShapeDtypeStruct((BATCH, NCLS), jnp.float32),
    mesh=plsc.VectorSubcoreMesh(core_axis_name="c", subcore_axis_name="s"),
    scratch_types=[
        pltpu.VMEM((2, CB, F), jnp.int32),
        pltpu.VMEM((V,), jnp.int32),
        pltpu.VMEM((2, CB, NCLS), jnp.float32),
        pltpu.VMEM((128,), jnp.float32),
        pltpu.SemaphoreType.DMA((2,)),
        pltpu.SemaphoreType.DMA((2,)),
    ],
    compiler_params=pltpu.CompilerParams(use_tc_tiling_on_sc=False,
                                         needs_layout_passes=False),
)(_sc_body)


def kernel(x, emb, W1, b1, W2, b2, Wout, bout):
    tbl, hw = _packed_table(W1, b1, W2, b2, Wout, bout, emb)
    return _sc_main(x, tbl, hw)
